# baseline (speedup 1.0000x reference)
import functools

import jax
import jax.numpy as jnp
from jax import lax
from jax.experimental import pallas as pl
from jax.experimental.pallas import tpu as pltpu, tpu_sc as plsc

ENV = 4
F = 64
B = 16384
U = 1000000
W128 = 128
PSUB = 499968
HT = 500032

NC, NS, L = 2, 16, 16
NW = NC * NS
RPW = B // NW
HB = RPW // 2
NGH = HB // L
IDXC = 128
NIDX = RPW // IDXC
NT = F // L

TCG = PSUB // 128 + 1

_f32 = jnp.float32
_i32 = jnp.int32


def _lane_bcast(v, k):
    idx = jnp.full((L, 1), k, _i32)
    dn = lax.GatherDimensionNumbers(
        offset_dims=(), collapsed_slice_dims=(0,), start_index_map=(0,))
    return lax.gather(v, idx, dn, (1,),
                      mode=lax.GatherScatterMode.PROMISE_IN_BOUNDS)


def _sigmoid(x):
    return 1.0 / (1.0 + jnp.exp(-x))


def _log1to4(s):
    w = (s - 1.0) / (s + 1.0)
    w2 = w * w
    ln = 2.0 * w * (1.0 + w2 * (1.0 / 3.0 + w2 * (0.2 + w2 * (1.0 / 7.0))))
    return ln + s * jnp.exp(-ln) - 1.0


def _tc_body(a_ref, b_ref, out_ref):
    out_ref[:, 0:F] = jnp.transpose(a_ref[...])
    out_ref[:, F:W128] = jnp.transpose(b_ref[...])


_tc_pack = pl.pallas_call(
    _tc_body,
    grid=(TCG,),
    in_specs=[
        pl.BlockSpec((F, 128), lambda j: (0, j)),
        pl.BlockSpec((F, 128), lambda j: (0, j + PSUB // 128)),
    ],
    out_specs=pl.BlockSpec((128, W128), lambda j: (j, 0)),
    out_shape=jax.ShapeDtypeStruct((HT, W128), _f32),
)


_mesh = plsc.VectorSubcoreMesh(core_axis_name="c", subcore_axis_name="s")


@functools.partial(
    pl.kernel,
    mesh=_mesh,
    compiler_params=pltpu.CompilerParams(
        needs_layout_passes=False, use_tc_tiling_on_sc=False),
    out_type=(
        jax.ShapeDtypeStruct((B,), _f32),
        jax.ShapeDtypeStruct((B,), _f32),
        jax.ShapeDtypeStruct((B * ENV,), _f32),
    ),
    scratch_types=[
        pltpu.VMEM((NIDX, IDXC), _i32),
        pltpu.VMEM((NIDX, IDXC), _i32),
        pltpu.VMEM((NIDX, IDXC), _i32),
        pltpu.VMEM((NIDX, IDXC), _i32),
        pltpu.VMEM((NIDX, IDXC), _i32),
        pltpu.VMEM((NIDX, IDXC), _i32),
        pltpu.VMEM((RPW,), _i32),
        pltpu.VMEM((RPW,), _i32),
        pltpu.VMEM((RPW,), _i32),
        pltpu.VMEM((HB, W128), _f32),
        pltpu.VMEM((HB, W128), _f32),
        pltpu.VMEM((HB, W128), _f32),
        pltpu.VMEM((ENV, F), _f32),
        pltpu.VMEM((L,), _f32),
        pltpu.VMEM((RPW,), _f32),
        pltpu.VMEM((RPW,), _f32),
        pltpu.VMEM((RPW * ENV,), _f32),
        pltpu.SemaphoreType.DMA,
    ],
)
def _sc_forward(u2d, i2d, e2d, wui, wii, wue, wie, wenv_h, clfw_h, clfb_h,
                o_inv, o_env, o_cls,
                idxu_v, idxi_v, idxe_v, hidu_v, hidi_v, hide_v,
                pofu_v, pofi_v, pofe_v, rows_u, rows_i, rows_e, clfw_v,
                clfb_v, invs_v, envsc_v, envout_v, sem):
    cid = lax.axis_index("c")
    sid = lax.axis_index("s")
    wid = sid * NC + cid
    base = wid * RPW
    brow = wid * NIDX

    pltpu.sync_copy(u2d.at[pl.ds(brow, NIDX)], idxu_v)
    pltpu.sync_copy(i2d.at[pl.ds(brow, NIDX)], idxi_v)
    pltpu.sync_copy(e2d.at[pl.ds(brow, NIDX)], idxe_v)
    pltpu.sync_copy(clfw_h, clfw_v)
    pltpu.sync_copy(clfb_h, clfb_v)

    for ids, hid, pof, thr, sub in ((idxu_v, hidu_v, pofu_v, HT, PSUB),
                                    (idxi_v, hidi_v, pofi_v, HT, PSUB),
                                    (idxe_v, hide_v, pofe_v, 2, 2)):
        def split_ids(j, _, ids=ids, hid=hid, pof=pof, thr=thr, sub=sub):
            jj = j // (IDXC // L)
            oo = (j % (IDXC // L)) * L
            v = ids[jj, pl.ds(oo, L)]
            ge = v >= thr
            hid[jj, pl.ds(oo, L)] = v - jnp.where(ge, sub, 0)
            pof[pl.ds(j * L, L)] = jnp.where(ge, F, 0)
            return 0
        lax.fori_loop(0, RPW // L, split_ids, 0)

    def gather_half(tab, hid_v, dst, h):
        cps = []
        for j in range(HB // IDXC):
            cps.append(pltpu.async_copy(
                tab.at[hid_v.at[h * (HB // IDXC) + j]],
                dst.at[pl.ds(j * IDXC, IDXC)], sem))
        return cps

    iota = lax.iota(_i32, L)
    masks = [iota == r for r in range(L)]
    bvec = clfb_v[...]
    w = [[clfw_v[k, pl.ds(t * L, L)] for t in range(NT)] for k in range(ENV)]

    def lane_sum_into(acc, vec, r):
        tot = _lane_bcast(plsc.cumsum(vec), L - 1)
        return jnp.where(masks[r], tot, acc)

    for h in range(2):
        hbase = h * HB

        cps = (gather_half(wui, hidu_v, rows_u, h)
               + gather_half(wii, hidi_v, rows_i, h))
        for cp in cps:
            cp.wait()

        def group1(g, _):
            z = jnp.zeros((L,), _f32)
            a0, a1, a2, a3, a4 = z, z, z, z, z
            pu16 = pofu_v[pl.ds(hbase + g * L, L)]
            pi16 = pofi_v[pl.ds(hbase + g * L, L)]
            for r in range(L):
                row = g * L + r
                pu = pu16[r]
                pi = pi16[r]
                pt = [rows_u[row, pl.ds(pu + t * L, L)]
                      * rows_i[row, pl.ds(pi + t * L, L)] for t in range(NT)]
                s = (pt[0] + pt[1]) + (pt[2] + pt[3])
                a0 = lane_sum_into(a0, s, r)
                q = [(pt[0] * w[k][0] + pt[1] * w[k][1])
                     + (pt[2] * w[k][2] + pt[3] * w[k][3])
                     for k in range(ENV)]
                a1 = lane_sum_into(a1, q[0], r)
                a2 = lane_sum_into(a2, q[1], r)
                a3 = lane_sum_into(a3, q[2], r)
                a4 = lane_sum_into(a4, q[3], r)

            invs_v[pl.ds(hbase + g * L, L)] = _sigmoid(a0)

            l0 = a1 + _lane_bcast(bvec, 0)
            l1 = a2 + _lane_bcast(bvec, 1)
            l2 = a3 + _lane_bcast(bvec, 2)
            l3 = a4 + _lane_bcast(bvec, 3)
            m = jnp.maximum(jnp.maximum(l0, l1), jnp.maximum(l2, l3))
            e0 = jnp.exp(l0 - m)
            e1 = jnp.exp(l1 - m)
            e2 = jnp.exp(l2 - m)
            e3 = jnp.exp(l3 - m)
            ssum = (e0 + e1) + (e2 + e3)
            lse = m + _log1to4(ssum)
            rl4 = (hbase + g * L + iota) * ENV
            plsc.store_scatter(envout_v, [rl4], l0 - lse)
            plsc.store_scatter(envout_v, [rl4 + 1], l1 - lse)
            plsc.store_scatter(envout_v, [rl4 + 2], l2 - lse)
            plsc.store_scatter(envout_v, [rl4 + 3], l3 - lse)
            return 0

        lax.fori_loop(0, NGH, group1, 0)

        cps = (gather_half(wue, hidu_v, rows_u, h)
               + gather_half(wie, hidi_v, rows_i, h)
               + gather_half(wenv_h, hide_v, rows_e, h))
        for cp in cps:
            cp.wait()

        def group2(g, _):
            acc = jnp.zeros((L,), _f32)
            pu16 = pofu_v[pl.ds(hbase + g * L, L)]
            pi16 = pofi_v[pl.ds(hbase + g * L, L)]
            pe16 = pofe_v[pl.ds(hbase + g * L, L)]
            for r in range(L):
                row = g * L + r
                pu = pu16[r]
                pi = pi16[r]
                pe = pe16[r]
                pt = [rows_u[row, pl.ds(pu + t * L, L)]
                      * rows_i[row, pl.ds(pi + t * L, L)]
                      * rows_e[row, pl.ds(pe + t * L, L)] for t in range(NT)]
                s = (pt[0] + pt[1]) + (pt[2] + pt[3])
                acc = lane_sum_into(acc, s, r)
            mid = _sigmoid(acc)
            gg = pl.ds(hbase + g * L, L)
            envsc_v[gg] = invs_v[gg] * mid
            return 0

        lax.fori_loop(0, NGH, group2, 0)

    pltpu.sync_copy(invs_v, o_inv.at[pl.ds(base, RPW)])
    pltpu.sync_copy(envsc_v, o_env.at[pl.ds(base, RPW)])
    pltpu.sync_copy(envout_v, o_cls.at[pl.ds(base * ENV, RPW * ENV)])


def kernel(users_id, items_id, envs_id, alpha, W_user_inv, W_item_inv,
           W_user_env, W_item_env, W_env, clf_W, clf_b):
    del alpha
    u2d = users_id.reshape(B // IDXC, IDXC)
    i2d = items_id.reshape(B // IDXC, IDXC)
    e2d = envs_id.reshape(B // IDXC, IDXC)
    clfb = jnp.zeros((L,), _f32).at[:ENV].set(clf_b)
    wenv_p = jnp.concatenate([W_env[:ENV // 2], W_env[ENV // 2:]], axis=1)
    inv_s, env_s, env_out = _sc_forward(
        u2d, i2d, e2d,
        _tc_pack(W_user_inv.T, W_user_inv.T),
        _tc_pack(W_item_inv.T, W_item_inv.T),
        _tc_pack(W_user_env.T, W_user_env.T),
        _tc_pack(W_item_env.T, W_item_env.T),
        wenv_p, clf_W, clfb)
    return inv_s, env_s, env_out.reshape(B, ENV)


# speedup vs baseline: 7.2632x; 7.2632x over previous
import functools

import jax
import jax.numpy as jnp
from jax import lax
from jax.experimental import pallas as pl
from jax.experimental.pallas import tpu as pltpu, tpu_sc as plsc

ENV = 4
F = 64
B = 16384
U = 1000000
W128 = 128
PSUB = 499712
HT = 500288
TCB = 8192

NC, NS, L = 2, 16, 16
NW = NC * NS
RPW = B // NW
HB = RPW // 2
NGH = HB // L
IDXC = 128
NIDX = RPW // IDXC
NT = F // L

TCG = PSUB // TCB + 1

_f32 = jnp.float32
_i32 = jnp.int32


def _lane_bcast(v, k):
    idx = jnp.full((L, 1), k, _i32)
    dn = lax.GatherDimensionNumbers(
        offset_dims=(), collapsed_slice_dims=(0,), start_index_map=(0,))
    return lax.gather(v, idx, dn, (1,),
                      mode=lax.GatherScatterMode.PROMISE_IN_BOUNDS)


def _sigmoid(x):
    return 1.0 / (1.0 + jnp.exp(-x))


def _log1to4(s):
    w = (s - 1.0) / (s + 1.0)
    w2 = w * w
    ln = 2.0 * w * (1.0 + w2 * (1.0 / 3.0 + w2 * (0.2 + w2 * (1.0 / 7.0))))
    return ln + s * jnp.exp(-ln) - 1.0


def _tc_body(a_ref, b_ref, out_ref):
    out_ref[:, 0:F] = jnp.transpose(a_ref[...])
    out_ref[:, F:W128] = jnp.transpose(b_ref[...])


_tc_pack = pl.pallas_call(
    _tc_body,
    grid=(TCG,),
    in_specs=[
        pl.BlockSpec((F, TCB), lambda j: (0, j)),
        pl.BlockSpec((F, TCB), lambda j: (0, j + PSUB // TCB)),
    ],
    out_specs=pl.BlockSpec((TCB, W128), lambda j: (j, 0)),
    out_shape=jax.ShapeDtypeStruct((HT, W128), _f32),
)


_mesh = plsc.VectorSubcoreMesh(core_axis_name="c", subcore_axis_name="s")


@functools.partial(
    pl.kernel,
    mesh=_mesh,
    compiler_params=pltpu.CompilerParams(
        needs_layout_passes=False, use_tc_tiling_on_sc=False),
    out_type=(
        jax.ShapeDtypeStruct((B,), _f32),
        jax.ShapeDtypeStruct((B,), _f32),
        jax.ShapeDtypeStruct((B * ENV,), _f32),
    ),
    scratch_types=[
        pltpu.VMEM((NIDX, IDXC), _i32),
        pltpu.VMEM((NIDX, IDXC), _i32),
        pltpu.VMEM((NIDX, IDXC), _i32),
        pltpu.VMEM((NIDX, IDXC), _i32),
        pltpu.VMEM((NIDX, IDXC), _i32),
        pltpu.VMEM((NIDX, IDXC), _i32),
        pltpu.VMEM((RPW,), _i32),
        pltpu.VMEM((RPW,), _i32),
        pltpu.VMEM((RPW,), _i32),
        pltpu.VMEM((HB, W128), _f32),
        pltpu.VMEM((HB, W128), _f32),
        pltpu.VMEM((HB, W128), _f32),
        pltpu.VMEM((ENV, F), _f32),
        pltpu.VMEM((L,), _f32),
        pltpu.VMEM((RPW,), _f32),
        pltpu.VMEM((RPW,), _f32),
        pltpu.VMEM((RPW * ENV,), _f32),
        pltpu.SemaphoreType.DMA,
    ],
)
def _sc_forward(u2d, i2d, e2d, wui, wii, wue, wie, wenv_h, clfw_h, clfb_h,
                o_inv, o_env, o_cls,
                idxu_v, idxi_v, idxe_v, hidu_v, hidi_v, hide_v,
                pofu_v, pofi_v, pofe_v, rows_u, rows_i, rows_e, clfw_v,
                clfb_v, invs_v, envsc_v, envout_v, sem):
    cid = lax.axis_index("c")
    sid = lax.axis_index("s")
    wid = sid * NC + cid
    base = wid * RPW
    brow = wid * NIDX

    pltpu.sync_copy(u2d.at[pl.ds(brow, NIDX)], idxu_v)
    pltpu.sync_copy(i2d.at[pl.ds(brow, NIDX)], idxi_v)
    pltpu.sync_copy(e2d.at[pl.ds(brow, NIDX)], idxe_v)
    pltpu.sync_copy(clfw_h, clfw_v)
    pltpu.sync_copy(clfb_h, clfb_v)

    for ids, hid, pof, thr, sub in ((idxu_v, hidu_v, pofu_v, HT, PSUB),
                                    (idxi_v, hidi_v, pofi_v, HT, PSUB),
                                    (idxe_v, hide_v, pofe_v, 2, 2)):
        def split_ids(j, _, ids=ids, hid=hid, pof=pof, thr=thr, sub=sub):
            jj = j // (IDXC // L)
            oo = (j % (IDXC // L)) * L
            v = ids[jj, pl.ds(oo, L)]
            ge = v >= thr
            hid[jj, pl.ds(oo, L)] = v - jnp.where(ge, sub, 0)
            pof[pl.ds(j * L, L)] = jnp.where(ge, F, 0)
            return 0
        lax.fori_loop(0, RPW // L, split_ids, 0)

    def gather_half(tab, hid_v, dst, h):
        cps = []
        for j in range(HB // IDXC):
            cps.append(pltpu.async_copy(
                tab.at[hid_v.at[h * (HB // IDXC) + j]],
                dst.at[pl.ds(j * IDXC, IDXC)], sem))
        return cps

    iota = lax.iota(_i32, L)
    masks = [iota == r for r in range(L)]
    bvec = clfb_v[...]
    w = [[clfw_v[k, pl.ds(t * L, L)] for t in range(NT)] for k in range(ENV)]

    def lane_sum_into(acc, vec, r):
        tot = _lane_bcast(plsc.cumsum(vec), L - 1)
        return jnp.where(masks[r], tot, acc)

    for h in range(2):
        hbase = h * HB

        cps = (gather_half(wui, hidu_v, rows_u, h)
               + gather_half(wii, hidi_v, rows_i, h))
        for cp in cps:
            cp.wait()

        def group1(g, _):
            z = jnp.zeros((L,), _f32)
            a0, a1, a2, a3, a4 = z, z, z, z, z
            pu16 = pofu_v[pl.ds(hbase + g * L, L)]
            pi16 = pofi_v[pl.ds(hbase + g * L, L)]
            for r in range(L):
                row = g * L + r
                pu = pu16[r]
                pi = pi16[r]
                pt = [rows_u[row, pl.ds(pu + t * L, L)]
                      * rows_i[row, pl.ds(pi + t * L, L)] for t in range(NT)]
                s = (pt[0] + pt[1]) + (pt[2] + pt[3])
                a0 = lane_sum_into(a0, s, r)
                q = [(pt[0] * w[k][0] + pt[1] * w[k][1])
                     + (pt[2] * w[k][2] + pt[3] * w[k][3])
                     for k in range(ENV)]
                a1 = lane_sum_into(a1, q[0], r)
                a2 = lane_sum_into(a2, q[1], r)
                a3 = lane_sum_into(a3, q[2], r)
                a4 = lane_sum_into(a4, q[3], r)

            invs_v[pl.ds(hbase + g * L, L)] = _sigmoid(a0)

            l0 = a1 + _lane_bcast(bvec, 0)
            l1 = a2 + _lane_bcast(bvec, 1)
            l2 = a3 + _lane_bcast(bvec, 2)
            l3 = a4 + _lane_bcast(bvec, 3)
            m = jnp.maximum(jnp.maximum(l0, l1), jnp.maximum(l2, l3))
            e0 = jnp.exp(l0 - m)
            e1 = jnp.exp(l1 - m)
            e2 = jnp.exp(l2 - m)
            e3 = jnp.exp(l3 - m)
            ssum = (e0 + e1) + (e2 + e3)
            lse = m + _log1to4(ssum)
            rl4 = (hbase + g * L + iota) * ENV
            plsc.store_scatter(envout_v, [rl4], l0 - lse)
            plsc.store_scatter(envout_v, [rl4 + 1], l1 - lse)
            plsc.store_scatter(envout_v, [rl4 + 2], l2 - lse)
            plsc.store_scatter(envout_v, [rl4 + 3], l3 - lse)
            return 0

        lax.fori_loop(0, NGH, group1, 0)

        cps = (gather_half(wue, hidu_v, rows_u, h)
               + gather_half(wie, hidi_v, rows_i, h)
               + gather_half(wenv_h, hide_v, rows_e, h))
        for cp in cps:
            cp.wait()

        def group2(g, _):
            acc = jnp.zeros((L,), _f32)
            pu16 = pofu_v[pl.ds(hbase + g * L, L)]
            pi16 = pofi_v[pl.ds(hbase + g * L, L)]
            pe16 = pofe_v[pl.ds(hbase + g * L, L)]
            for r in range(L):
                row = g * L + r
                pu = pu16[r]
                pi = pi16[r]
                pe = pe16[r]
                pt = [rows_u[row, pl.ds(pu + t * L, L)]
                      * rows_i[row, pl.ds(pi + t * L, L)]
                      * rows_e[row, pl.ds(pe + t * L, L)] for t in range(NT)]
                s = (pt[0] + pt[1]) + (pt[2] + pt[3])
                acc = lane_sum_into(acc, s, r)
            mid = _sigmoid(acc)
            gg = pl.ds(hbase + g * L, L)
            envsc_v[gg] = invs_v[gg] * mid
            return 0

        lax.fori_loop(0, NGH, group2, 0)

    pltpu.sync_copy(invs_v, o_inv.at[pl.ds(base, RPW)])
    pltpu.sync_copy(envsc_v, o_env.at[pl.ds(base, RPW)])
    pltpu.sync_copy(envout_v, o_cls.at[pl.ds(base * ENV, RPW * ENV)])


def kernel(users_id, items_id, envs_id, alpha, W_user_inv, W_item_inv,
           W_user_env, W_item_env, W_env, clf_W, clf_b):
    del alpha
    u2d = users_id.reshape(B // IDXC, IDXC)
    i2d = items_id.reshape(B // IDXC, IDXC)
    e2d = envs_id.reshape(B // IDXC, IDXC)
    clfb = jnp.zeros((L,), _f32).at[:ENV].set(clf_b)
    wenv_p = jnp.concatenate([W_env[:ENV // 2], W_env[ENV // 2:]], axis=1)
    inv_s, env_s, env_out = _sc_forward(
        u2d, i2d, e2d,
        _tc_pack(W_user_inv.T, W_user_inv.T),
        _tc_pack(W_item_inv.T, W_item_inv.T),
        _tc_pack(W_user_env.T, W_user_env.T),
        _tc_pack(W_item_env.T, W_item_env.T),
        wenv_p, clf_W, clfb)
    return inv_s, env_s, env_out.reshape(B, ENV)


# speedup vs baseline: 8.3021x; 1.1430x over previous
import functools

import jax
import jax.numpy as jnp
from jax import lax
from jax.experimental import pallas as pl
from jax.experimental.pallas import tpu as pltpu, tpu_sc as plsc

ENV = 4
F = 64
B = 16384
U = 1000000
W128 = 128
PSUB = 499712
HT = U - PSUB
TCB = 8192
TCG = PSUB // TCB + 1

NC, NS, L = 2, 16, 16
NW = NC * NS
RPW = B // NW
NG = RPW // L
IDXC = 128
NIDX = RPW // IDXC
NT = F // L

_f32 = jnp.float32
_i32 = jnp.int32


def _lane_bcast(v, k):
    idx = jnp.full((L, 1), k, _i32)
    dn = lax.GatherDimensionNumbers(
        offset_dims=(), collapsed_slice_dims=(0,), start_index_map=(0,))
    return lax.gather(v, idx, dn, (1,),
                      mode=lax.GatherScatterMode.PROMISE_IN_BOUNDS)


def _sigmoid(x):
    return 1.0 / (1.0 + jnp.exp(-x))


def _log1to4(s):
    w = (s - 1.0) / (s + 1.0)
    w2 = w * w
    ln = 2.0 * w * (1.0 + w2 * (1.0 / 3.0 + w2 * (0.2 + w2 * (1.0 / 7.0))))
    return ln + s * jnp.exp(-ln) - 1.0


def _tc_body(a_ref, b_ref, out_ref):
    out_ref[:, 0:F] = jnp.transpose(a_ref[...])
    out_ref[:, F:W128] = jnp.transpose(b_ref[...])


_tc_pack = pl.pallas_call(
    _tc_body,
    grid=(TCG,),
    in_specs=[
        pl.BlockSpec((F, TCB), lambda j: (0, j)),
        pl.BlockSpec((F, TCB), lambda j: (0, j + PSUB // TCB)),
    ],
    out_specs=pl.BlockSpec((TCB, W128), lambda j: (j, 0)),
    out_shape=jax.ShapeDtypeStruct((HT, W128), _f32),
)


_mesh = plsc.VectorSubcoreMesh(core_axis_name="c", subcore_axis_name="s")


@functools.partial(
    pl.kernel,
    mesh=_mesh,
    compiler_params=pltpu.CompilerParams(
        needs_layout_passes=False, use_tc_tiling_on_sc=False),
    out_type=(
        jax.ShapeDtypeStruct((B,), _f32),
        jax.ShapeDtypeStruct((B,), _f32),
        jax.ShapeDtypeStruct((B * ENV,), _f32),
    ),
    scratch_types=[
        pltpu.VMEM((NIDX, IDXC), _i32),
        pltpu.VMEM((NIDX, IDXC), _i32),
        pltpu.VMEM((NIDX, IDXC), _i32),
        pltpu.VMEM((NIDX, IDXC), _i32),
        pltpu.VMEM((NIDX, IDXC), _i32),
        pltpu.VMEM((NIDX, IDXC), _i32),
        pltpu.VMEM((RPW, F), _f32),
        pltpu.VMEM((RPW, F), _f32),
        pltpu.VMEM((RPW, F), _f32),
        pltpu.VMEM((ENV, F), _f32),
        pltpu.VMEM((L,), _f32),
        pltpu.VMEM((RPW,), _f32),
        pltpu.VMEM((RPW,), _f32),
        pltpu.VMEM((RPW * ENV,), _f32),
        pltpu.SemaphoreType.DMA,
    ],
)
def _sc_forward(u2d, i2d, e2d, wui, wii, wue, wie, wenv_h, clfw_h, clfb_h,
                o_inv, o_env, o_cls,
                idxu_v, idxi_v, idxe_v, hidu_v, hidi_v, hide_v,
                rows_u, rows_i, rows_e, clfw_v,
                clfb_v, invs_v, envsc_v, envout_v, sem):
    cid = lax.axis_index("c")
    sid = lax.axis_index("s")
    wid = sid * NC + cid
    base = wid * RPW
    brow = wid * NIDX

    pltpu.sync_copy(u2d.at[pl.ds(brow, NIDX)], idxu_v)
    pltpu.sync_copy(i2d.at[pl.ds(brow, NIDX)], idxi_v)
    pltpu.sync_copy(e2d.at[pl.ds(brow, NIDX)], idxe_v)
    pltpu.sync_copy(clfw_h, clfw_v)
    pltpu.sync_copy(clfb_h, clfb_v)

    for ids, hid, thr, sub in ((idxu_v, hidu_v, HT, PSUB),
                               (idxi_v, hidi_v, HT, PSUB),
                               (idxe_v, hide_v, 2, 2)):
        def split_ids(j, _, ids=ids, hid=hid, thr=thr, sub=sub):
            jj = j // (IDXC // L)
            oo = (j % (IDXC // L)) * L
            v = ids[jj, pl.ds(oo, L)]
            ge = v >= thr
            hid[jj, pl.ds(oo, L)] = (
                2 * (v - jnp.where(ge, sub, 0)) + jnp.where(ge, 1, 0))
            return 0
        lax.fori_loop(0, RPW // L, split_ids, 0)

    def gather_rows(tab, hid_v, dst):
        cps = []
        for j in range(NIDX):
            cps.append(pltpu.async_copy(
                tab.at[hid_v.at[j]], dst.at[pl.ds(j * IDXC, IDXC)], sem))
        return cps

    iota = lax.iota(_i32, L)
    masks = [iota == r for r in range(L)]
    bvec = clfb_v[...]
    w = [[clfw_v[k, pl.ds(t * L, L)] for t in range(NT)] for k in range(ENV)]

    def lane_sum_into(acc, vec, r):
        tot = _lane_bcast(plsc.cumsum(vec), L - 1)
        return jnp.where(masks[r], tot, acc)

    cps = gather_rows(wui, hidu_v, rows_u) + gather_rows(wii, hidi_v, rows_i)
    for cp in cps:
        cp.wait()

    def group1(g, _):
        z = jnp.zeros((L,), _f32)
        a0, a1, a2, a3, a4 = z, z, z, z, z
        for r in range(L):
            row = g * L + r
            pt = [rows_u[row, pl.ds(t * L, L)] * rows_i[row, pl.ds(t * L, L)]
                  for t in range(NT)]
            s = (pt[0] + pt[1]) + (pt[2] + pt[3])
            a0 = lane_sum_into(a0, s, r)
            q = [(pt[0] * w[k][0] + pt[1] * w[k][1])
                 + (pt[2] * w[k][2] + pt[3] * w[k][3]) for k in range(ENV)]
            a1 = lane_sum_into(a1, q[0], r)
            a2 = lane_sum_into(a2, q[1], r)
            a3 = lane_sum_into(a3, q[2], r)
            a4 = lane_sum_into(a4, q[3], r)

        invs_v[pl.ds(g * L, L)] = _sigmoid(a0)

        l0 = a1 + _lane_bcast(bvec, 0)
        l1 = a2 + _lane_bcast(bvec, 1)
        l2 = a3 + _lane_bcast(bvec, 2)
        l3 = a4 + _lane_bcast(bvec, 3)
        m = jnp.maximum(jnp.maximum(l0, l1), jnp.maximum(l2, l3))
        e0 = jnp.exp(l0 - m)
        e1 = jnp.exp(l1 - m)
        e2 = jnp.exp(l2 - m)
        e3 = jnp.exp(l3 - m)
        ssum = (e0 + e1) + (e2 + e3)
        lse = m + _log1to4(ssum)
        rl4 = (g * L + iota) * ENV
        plsc.store_scatter(envout_v, [rl4], l0 - lse)
        plsc.store_scatter(envout_v, [rl4 + 1], l1 - lse)
        plsc.store_scatter(envout_v, [rl4 + 2], l2 - lse)
        plsc.store_scatter(envout_v, [rl4 + 3], l3 - lse)
        return 0

    lax.fori_loop(0, NG, group1, 0)

    cps = (gather_rows(wue, hidu_v, rows_u)
           + gather_rows(wie, hidi_v, rows_i)
           + gather_rows(wenv_h, hide_v, rows_e))
    for cp in cps:
        cp.wait()

    def group2(g, _):
        acc = jnp.zeros((L,), _f32)
        for r in range(L):
            row = g * L + r
            pt = [rows_u[row, pl.ds(t * L, L)] * rows_i[row, pl.ds(t * L, L)]
                  * rows_e[row, pl.ds(t * L, L)] for t in range(NT)]
            s = (pt[0] + pt[1]) + (pt[2] + pt[3])
            acc = lane_sum_into(acc, s, r)
        mid = _sigmoid(acc)
        gg = pl.ds(g * L, L)
        envsc_v[gg] = invs_v[gg] * mid
        return 0

    lax.fori_loop(0, NG, group2, 0)

    pltpu.sync_copy(invs_v, o_inv.at[pl.ds(base, RPW)])
    pltpu.sync_copy(envsc_v, o_env.at[pl.ds(base, RPW)])
    pltpu.sync_copy(envout_v, o_cls.at[pl.ds(base * ENV, RPW * ENV)])


def kernel(users_id, items_id, envs_id, alpha, W_user_inv, W_item_inv,
           W_user_env, W_item_env, W_env, clf_W, clf_b):
    del alpha
    u2d = users_id.reshape(B // IDXC, IDXC)
    i2d = items_id.reshape(B // IDXC, IDXC)
    e2d = envs_id.reshape(B // IDXC, IDXC)
    clfb = jnp.zeros((L,), _f32).at[:ENV].set(clf_b)
    wenv_p = jnp.concatenate(
        [W_env[:ENV // 2], W_env[ENV // 2:]], axis=1).reshape(ENV, F)

    def pack(t):
        return _tc_pack(t.T, t.T).reshape(2 * HT, F)

    inv_s, env_s, env_out = _sc_forward(
        u2d, i2d, e2d,
        pack(W_user_inv), pack(W_item_inv),
        pack(W_user_env), pack(W_item_env),
        wenv_p, clf_W, clfb)
    return inv_s, env_s, env_out.reshape(B, ENV)


# speedup vs baseline: 8.5684x; 1.0321x over previous
import functools

import jax
import jax.numpy as jnp
from jax import lax
from jax.experimental import pallas as pl
from jax.experimental.pallas import tpu as pltpu, tpu_sc as plsc

ENV = 4
F = 64
B = 16384
U = 1000000
W128 = 128
PSUB = 491520
HT = U - PSUB
TCB = 16384
TCG = (HT + TCB - 1) // TCB

NC, NS, L = 2, 16, 16
NW = NC * NS
RPW = B // NW
NG = RPW // L
IDXC = 128
NIDX = RPW // IDXC
NT = F // L

_f32 = jnp.float32
_i32 = jnp.int32


def _lane_bcast(v, k):
    idx = jnp.full((L, 1), k, _i32)
    dn = lax.GatherDimensionNumbers(
        offset_dims=(), collapsed_slice_dims=(0,), start_index_map=(0,))
    return lax.gather(v, idx, dn, (1,),
                      mode=lax.GatherScatterMode.PROMISE_IN_BOUNDS)


def _sigmoid(x):
    return 1.0 / (1.0 + jnp.exp(-x))


def _log1to4(s):
    w = (s - 1.0) / (s + 1.0)
    w2 = w * w
    ln = 2.0 * w * (1.0 + w2 * (1.0 / 3.0 + w2 * (0.2 + w2 * (1.0 / 7.0))))
    return ln + s * jnp.exp(-ln) - 1.0


def _tc_body(a_ref, b_ref, out_ref):
    out_ref[:, 0:F] = jnp.transpose(a_ref[...])
    out_ref[:, F:W128] = jnp.transpose(b_ref[...])


_tc_pack = pl.pallas_call(
    _tc_body,
    grid=(TCG,),
    in_specs=[
        pl.BlockSpec((F, TCB), lambda j: (0, j)),
        pl.BlockSpec((F, TCB), lambda j: (0, j + PSUB // TCB)),
    ],
    out_specs=pl.BlockSpec((TCB, W128), lambda j: (j, 0)),
    out_shape=jax.ShapeDtypeStruct((HT, W128), _f32),
)


_mesh = plsc.VectorSubcoreMesh(core_axis_name="c", subcore_axis_name="s")


@functools.partial(
    pl.kernel,
    mesh=_mesh,
    compiler_params=pltpu.CompilerParams(
        needs_layout_passes=False, use_tc_tiling_on_sc=False),
    out_type=(
        jax.ShapeDtypeStruct((B,), _f32),
        jax.ShapeDtypeStruct((B,), _f32),
        jax.ShapeDtypeStruct((B * ENV,), _f32),
    ),
    scratch_types=[
        pltpu.VMEM((NIDX, IDXC), _i32),
        pltpu.VMEM((NIDX, IDXC), _i32),
        pltpu.VMEM((NIDX, IDXC), _i32),
        pltpu.VMEM((NIDX, IDXC), _i32),
        pltpu.VMEM((NIDX, IDXC), _i32),
        pltpu.VMEM((NIDX, IDXC), _i32),
        pltpu.VMEM((RPW, F), _f32),
        pltpu.VMEM((RPW, F), _f32),
        pltpu.VMEM((RPW, F), _f32),
        pltpu.VMEM((ENV, F), _f32),
        pltpu.VMEM((L,), _f32),
        pltpu.VMEM((RPW,), _f32),
        pltpu.VMEM((RPW,), _f32),
        pltpu.VMEM((RPW * ENV,), _f32),
        pltpu.SemaphoreType.DMA,
    ],
)
def _sc_forward(u2d, i2d, e2d, wui, wii, wue, wie, wenv_h, clfw_h, clfb_h,
                o_inv, o_env, o_cls,
                idxu_v, idxi_v, idxe_v, hidu_v, hidi_v, hide_v,
                rows_u, rows_i, rows_e, clfw_v,
                clfb_v, invs_v, envsc_v, envout_v, sem):
    cid = lax.axis_index("c")
    sid = lax.axis_index("s")
    wid = sid * NC + cid
    base = wid * RPW
    brow = wid * NIDX

    pltpu.sync_copy(u2d.at[pl.ds(brow, NIDX)], idxu_v)
    pltpu.sync_copy(i2d.at[pl.ds(brow, NIDX)], idxi_v)
    pltpu.sync_copy(e2d.at[pl.ds(brow, NIDX)], idxe_v)
    pltpu.sync_copy(clfw_h, clfw_v)
    pltpu.sync_copy(clfb_h, clfb_v)

    for ids, hid, thr, sub in ((idxu_v, hidu_v, HT, PSUB),
                               (idxi_v, hidi_v, HT, PSUB),
                               (idxe_v, hide_v, 2, 2)):
        def split_ids(j, _, ids=ids, hid=hid, thr=thr, sub=sub):
            jj = j // (IDXC // L)
            oo = (j % (IDXC // L)) * L
            v = ids[jj, pl.ds(oo, L)]
            ge = v >= thr
            hid[jj, pl.ds(oo, L)] = (
                2 * (v - jnp.where(ge, sub, 0)) + jnp.where(ge, 1, 0))
            return 0
        lax.fori_loop(0, RPW // L, split_ids, 0)

    def gather_rows(tab, hid_v, dst):
        cps = []
        for j in range(NIDX):
            cps.append(pltpu.async_copy(
                tab.at[hid_v.at[j]], dst.at[pl.ds(j * IDXC, IDXC)], sem))
        return cps

    iota = lax.iota(_i32, L)
    masks = [iota == r for r in range(L)]
    bvec = clfb_v[...]
    w = [[clfw_v[k, pl.ds(t * L, L)] for t in range(NT)] for k in range(ENV)]

    def lane_sum_into(acc, vec, r):
        tot = _lane_bcast(plsc.cumsum(vec), L - 1)
        return jnp.where(masks[r], tot, acc)

    def load_row_f32(ref, row):
        return [ref[row, pl.ds(t * L, L)] for t in range(NT)]

    cps = gather_rows(wui, hidu_v, rows_u) + gather_rows(wii, hidi_v, rows_i)
    for cp in cps:
        cp.wait()

    def group1(g, _):
        z = jnp.zeros((L,), _f32)
        a0, a1, a2, a3, a4 = z, z, z, z, z
        for r in range(L):
            row = g * L + r
            ut = load_row_f32(rows_u, row)
            it = load_row_f32(rows_i, row)
            pt = [ut[t] * it[t] for t in range(NT)]
            s = (pt[0] + pt[1]) + (pt[2] + pt[3])
            a0 = lane_sum_into(a0, s, r)
            q = [(pt[0] * w[k][0] + pt[1] * w[k][1])
                 + (pt[2] * w[k][2] + pt[3] * w[k][3]) for k in range(ENV)]
            a1 = lane_sum_into(a1, q[0], r)
            a2 = lane_sum_into(a2, q[1], r)
            a3 = lane_sum_into(a3, q[2], r)
            a4 = lane_sum_into(a4, q[3], r)

        invs_v[pl.ds(g * L, L)] = _sigmoid(a0)

        l0 = a1 + _lane_bcast(bvec, 0)
        l1 = a2 + _lane_bcast(bvec, 1)
        l2 = a3 + _lane_bcast(bvec, 2)
        l3 = a4 + _lane_bcast(bvec, 3)
        m = jnp.maximum(jnp.maximum(l0, l1), jnp.maximum(l2, l3))
        e0 = jnp.exp(l0 - m)
        e1 = jnp.exp(l1 - m)
        e2 = jnp.exp(l2 - m)
        e3 = jnp.exp(l3 - m)
        ssum = (e0 + e1) + (e2 + e3)
        lse = m + _log1to4(ssum)
        rl4 = (g * L + iota) * ENV
        plsc.store_scatter(envout_v, [rl4], l0 - lse)
        plsc.store_scatter(envout_v, [rl4 + 1], l1 - lse)
        plsc.store_scatter(envout_v, [rl4 + 2], l2 - lse)
        plsc.store_scatter(envout_v, [rl4 + 3], l3 - lse)
        return 0

    lax.fori_loop(0, NG, group1, 0)

    cps = (gather_rows(wue, hidu_v, rows_u)
           + gather_rows(wie, hidi_v, rows_i)
           + gather_rows(wenv_h, hide_v, rows_e))
    for cp in cps:
        cp.wait()

    def group2(g, _):
        acc = jnp.zeros((L,), _f32)
        for r in range(L):
            row = g * L + r
            ut = load_row_f32(rows_u, row)
            it = load_row_f32(rows_i, row)
            et = load_row_f32(rows_e, row)
            pt = [ut[t] * it[t] * et[t] for t in range(NT)]
            s = (pt[0] + pt[1]) + (pt[2] + pt[3])
            acc = lane_sum_into(acc, s, r)
        mid = _sigmoid(acc)
        gg = pl.ds(g * L, L)
        envsc_v[gg] = invs_v[gg] * mid
        return 0

    lax.fori_loop(0, NG, group2, 0)

    pltpu.sync_copy(invs_v, o_inv.at[pl.ds(base, RPW)])
    pltpu.sync_copy(envsc_v, o_env.at[pl.ds(base, RPW)])
    pltpu.sync_copy(envout_v, o_cls.at[pl.ds(base * ENV, RPW * ENV)])


def kernel(users_id, items_id, envs_id, alpha, W_user_inv, W_item_inv,
           W_user_env, W_item_env, W_env, clf_W, clf_b):
    del alpha
    u2d = users_id.reshape(B // IDXC, IDXC)
    i2d = items_id.reshape(B // IDXC, IDXC)
    e2d = envs_id.reshape(B // IDXC, IDXC)
    clfb = jnp.zeros((L,), _f32).at[:ENV].set(clf_b)
    wenv_p = jnp.concatenate(
        [W_env[:ENV // 2], W_env[ENV // 2:]], axis=1).reshape(ENV, F)

    def pack(t):
        return _tc_pack(t.T, t.T).reshape(2 * HT, F)

    inv_s, env_s, env_out = _sc_forward(
        u2d, i2d, e2d,
        pack(W_user_inv), pack(W_item_inv),
        pack(W_user_env), pack(W_item_env),
        wenv_p, clf_W, clfb)
    return inv_s, env_s, env_out.reshape(B, ENV)


# speedup vs baseline: 8.7706x; 1.0236x over previous
import functools

import jax
import jax.numpy as jnp
from jax import lax
from jax.experimental import pallas as pl
from jax.experimental.pallas import tpu as pltpu, tpu_sc as plsc

ENV = 4
F = 64
B = 16384
U = 1000000
W128 = 128
PSUB = 491520
HT = U - PSUB
TCB = 16384
TCG = (HT + TCB - 1) // TCB

NC, NS, L = 2, 16, 16
NW = NC * NS
RPW = B // NW
NG = RPW // L
IDXC = 128
NIDX = RPW // IDXC
NT = F // L

_f32 = jnp.float32
_i32 = jnp.int32


def _lane_bcast(v, k):
    idx = jnp.full((L, 1), k, _i32)
    dn = lax.GatherDimensionNumbers(
        offset_dims=(), collapsed_slice_dims=(0,), start_index_map=(0,))
    return lax.gather(v, idx, dn, (1,),
                      mode=lax.GatherScatterMode.PROMISE_IN_BOUNDS)


def _sigmoid(x):
    return 1.0 / (1.0 + jnp.exp(-x))


def _log1to4(s):
    w = (s - 1.0) / (s + 1.0)
    w2 = w * w
    ln = 2.0 * w * (1.0 + w2 * (1.0 / 3.0 + w2 * (0.2 + w2 * (1.0 / 7.0))))
    return ln + s * jnp.exp(-ln) - 1.0


def _tc_body(a_ref, b_ref, out_ref):
    out_ref[:, 0:F] = jnp.transpose(a_ref[...])
    out_ref[:, F:W128] = jnp.transpose(b_ref[...])


_tc_pack = pl.pallas_call(
    _tc_body,
    grid=(TCG,),
    in_specs=[
        pl.BlockSpec((F, TCB), lambda j: (0, j)),
        pl.BlockSpec((F, TCB), lambda j: (0, j + PSUB // TCB)),
    ],
    out_specs=pl.BlockSpec((TCB, W128), lambda j: (j, 0)),
    out_shape=jax.ShapeDtypeStruct((HT, W128), _f32),
)


_mesh = plsc.VectorSubcoreMesh(core_axis_name="c", subcore_axis_name="s")
_params = pltpu.CompilerParams(
    needs_layout_passes=False, use_tc_tiling_on_sc=False)

_iota = None


def _split_ids(ids, hid, thr, sub):
    def body(j, _):
        jj = j // (IDXC // L)
        oo = (j % (IDXC // L)) * L
        v = ids[jj, pl.ds(oo, L)]
        ge = v >= thr
        hid[jj, pl.ds(oo, L)] = (
            2 * (v - jnp.where(ge, sub, 0)) + jnp.where(ge, 1, 0))
        return 0
    lax.fori_loop(0, RPW // L, body, 0)


def _gather_rows(tab, hid_v, dst, sem):
    cps = []
    for j in range(NIDX):
        cps.append(pltpu.async_copy(
            tab.at[hid_v.at[j]], dst.at[pl.ds(j * IDXC, IDXC)], sem))
    return cps


def _lane_sum_into(acc, vec, r, masks):
    tot = _lane_bcast(plsc.cumsum(vec), L - 1)
    return jnp.where(masks[r], tot, acc)


@functools.partial(
    pl.kernel,
    mesh=_mesh,
    compiler_params=_params,
    out_type=(
        jax.ShapeDtypeStruct((B,), _f32),
        jax.ShapeDtypeStruct((B * ENV,), _f32),
    ),
    scratch_types=[
        pltpu.VMEM((NIDX, IDXC), _i32),
        pltpu.VMEM((NIDX, IDXC), _i32),
        pltpu.VMEM((NIDX, IDXC), _i32),
        pltpu.VMEM((NIDX, IDXC), _i32),
        pltpu.VMEM((RPW, F), _f32),
        pltpu.VMEM((RPW, F), _f32),
        pltpu.VMEM((ENV, F), _f32),
        pltpu.VMEM((L,), _f32),
        pltpu.VMEM((RPW,), _f32),
        pltpu.VMEM((RPW * ENV,), _f32),
        pltpu.SemaphoreType.DMA,
    ],
)
def _sc_phase1(u2d, i2d, wui, wii, clfw_h, clfb_h,
               o_inv, o_cls,
               idxu_v, idxi_v, hidu_v, hidi_v, rows_u, rows_i, clfw_v,
               clfb_v, invs_v, envout_v, sem):
    cid = lax.axis_index("c")
    sid = lax.axis_index("s")
    wid = sid * NC + cid
    base = wid * RPW
    brow = wid * NIDX

    pltpu.sync_copy(u2d.at[pl.ds(brow, NIDX)], idxu_v)
    pltpu.sync_copy(i2d.at[pl.ds(brow, NIDX)], idxi_v)
    pltpu.sync_copy(clfw_h, clfw_v)
    pltpu.sync_copy(clfb_h, clfb_v)

    _split_ids(idxu_v, hidu_v, HT, PSUB)
    _split_ids(idxi_v, hidi_v, HT, PSUB)

    cps = (_gather_rows(wui, hidu_v, rows_u, sem)
           + _gather_rows(wii, hidi_v, rows_i, sem))
    for cp in cps:
        cp.wait()

    iota = lax.iota(_i32, L)
    masks = [iota == r for r in range(L)]
    bvec = clfb_v[...]
    w = [[clfw_v[k, pl.ds(t * L, L)] for t in range(NT)] for k in range(ENV)]

    def group1(g, _):
        z = jnp.zeros((L,), _f32)
        a0, a1, a2, a3, a4 = z, z, z, z, z
        for r in range(L):
            row = g * L + r
            pt = [rows_u[row, pl.ds(t * L, L)] * rows_i[row, pl.ds(t * L, L)]
                  for t in range(NT)]
            s = (pt[0] + pt[1]) + (pt[2] + pt[3])
            a0 = _lane_sum_into(a0, s, r, masks)
            q = [(pt[0] * w[k][0] + pt[1] * w[k][1])
                 + (pt[2] * w[k][2] + pt[3] * w[k][3]) for k in range(ENV)]
            a1 = _lane_sum_into(a1, q[0], r, masks)
            a2 = _lane_sum_into(a2, q[1], r, masks)
            a3 = _lane_sum_into(a3, q[2], r, masks)
            a4 = _lane_sum_into(a4, q[3], r, masks)

        invs_v[pl.ds(g * L, L)] = _sigmoid(a0)

        l0 = a1 + _lane_bcast(bvec, 0)
        l1 = a2 + _lane_bcast(bvec, 1)
        l2 = a3 + _lane_bcast(bvec, 2)
        l3 = a4 + _lane_bcast(bvec, 3)
        m = jnp.maximum(jnp.maximum(l0, l1), jnp.maximum(l2, l3))
        e0 = jnp.exp(l0 - m)
        e1 = jnp.exp(l1 - m)
        e2 = jnp.exp(l2 - m)
        e3 = jnp.exp(l3 - m)
        ssum = (e0 + e1) + (e2 + e3)
        lse = m + _log1to4(ssum)
        rl4 = (g * L + iota) * ENV
        plsc.store_scatter(envout_v, [rl4], l0 - lse)
        plsc.store_scatter(envout_v, [rl4 + 1], l1 - lse)
        plsc.store_scatter(envout_v, [rl4 + 2], l2 - lse)
        plsc.store_scatter(envout_v, [rl4 + 3], l3 - lse)
        return 0

    lax.fori_loop(0, NG, group1, 0)

    pltpu.sync_copy(invs_v, o_inv.at[pl.ds(base, RPW)])
    pltpu.sync_copy(envout_v, o_cls.at[pl.ds(base * ENV, RPW * ENV)])


@functools.partial(
    pl.kernel,
    mesh=_mesh,
    compiler_params=_params,
    out_type=jax.ShapeDtypeStruct((B,), _f32),
    scratch_types=[
        pltpu.VMEM((NIDX, IDXC), _i32),
        pltpu.VMEM((NIDX, IDXC), _i32),
        pltpu.VMEM((NIDX, IDXC), _i32),
        pltpu.VMEM((NIDX, IDXC), _i32),
        pltpu.VMEM((NIDX, IDXC), _i32),
        pltpu.VMEM((NIDX, IDXC), _i32),
        pltpu.VMEM((RPW, F), _f32),
        pltpu.VMEM((RPW, F), _f32),
        pltpu.VMEM((RPW, F), _f32),
        pltpu.VMEM((RPW,), _f32),
        pltpu.VMEM((RPW,), _f32),
        pltpu.SemaphoreType.DMA,
    ],
)
def _sc_phase2(u2d, i2d, e2d, wue, wie, wenv_h, invin,
               o_env,
               idxu_v, idxi_v, idxe_v, hidu_v, hidi_v, hide_v,
               rows_u, rows_i, rows_e, invs_v, envsc_v, sem):
    cid = lax.axis_index("c")
    sid = lax.axis_index("s")
    wid = sid * NC + cid
    base = wid * RPW
    brow = wid * NIDX

    pltpu.sync_copy(u2d.at[pl.ds(brow, NIDX)], idxu_v)
    pltpu.sync_copy(i2d.at[pl.ds(brow, NIDX)], idxi_v)
    pltpu.sync_copy(e2d.at[pl.ds(brow, NIDX)], idxe_v)
    pltpu.sync_copy(invin.at[pl.ds(base, RPW)], invs_v)

    _split_ids(idxu_v, hidu_v, HT, PSUB)
    _split_ids(idxi_v, hidi_v, HT, PSUB)
    _split_ids(idxe_v, hide_v, 2, 2)

    cps = (_gather_rows(wue, hidu_v, rows_u, sem)
           + _gather_rows(wie, hidi_v, rows_i, sem)
           + _gather_rows(wenv_h, hide_v, rows_e, sem))
    for cp in cps:
        cp.wait()

    iota = lax.iota(_i32, L)
    masks = [iota == r for r in range(L)]

    def group2(g, _):
        acc = jnp.zeros((L,), _f32)
        for r in range(L):
            row = g * L + r
            pt = [rows_u[row, pl.ds(t * L, L)] * rows_i[row, pl.ds(t * L, L)]
                  * rows_e[row, pl.ds(t * L, L)] for t in range(NT)]
            s = (pt[0] + pt[1]) + (pt[2] + pt[3])
            acc = _lane_sum_into(acc, s, r, masks)
        mid = _sigmoid(acc)
        gg = pl.ds(g * L, L)
        envsc_v[gg] = invs_v[gg] * mid
        return 0

    lax.fori_loop(0, NG, group2, 0)

    pltpu.sync_copy(envsc_v, o_env.at[pl.ds(base, RPW)])


def kernel(users_id, items_id, envs_id, alpha, W_user_inv, W_item_inv,
           W_user_env, W_item_env, W_env, clf_W, clf_b):
    del alpha
    u2d = users_id.reshape(B // IDXC, IDXC)
    i2d = items_id.reshape(B // IDXC, IDXC)
    e2d = envs_id.reshape(B // IDXC, IDXC)
    clfb = jnp.zeros((L,), _f32).at[:ENV].set(clf_b)
    wenv_p = jnp.concatenate(
        [W_env[:ENV // 2], W_env[ENV // 2:]], axis=1).reshape(ENV, F)

    def pack(t):
        return _tc_pack(t.T, t.T).reshape(2 * HT, F)

    inv_s, env_out = _sc_phase1(
        u2d, i2d, pack(W_user_inv), pack(W_item_inv), clf_W, clfb)
    env_s = _sc_phase2(
        u2d, i2d, e2d, pack(W_user_env), pack(W_item_env), wenv_p, inv_s)
    return inv_s, env_s, env_out.reshape(B, ENV)
